# baseline (speedup 1.0000x reference)
import jax
import jax.numpy as jnp
from jax.experimental import pallas as pl
from jax.experimental.pallas import tpu as pltpu

_T = 256
_TB = 512
_CH = 512


def _build_a_kernel(bounds_ref, rl_ref, cl_ref, at_ref, deg_ref):
    g = pl.program_id(0)
    n = at_ref.shape[0]
    half = n // 2

    row_iota = jax.lax.broadcasted_iota(jnp.int32, (_T, _CH), 0)
    col_iota = jax.lax.broadcasted_iota(jnp.int32, (n, _CH), 0)

    junk = jnp.concatenate(
        [pltpu.matmul_pop(0, (half, _T), jnp.float32, m) for m in (0, 1)],
        axis=0)
    at_ref[:, :_T] = junk.astype(at_ref.dtype)

    for t in range(2):
        i = 2 * g + t
        k0 = bounds_ref[i] // _CH
        k1 = (bounds_ref[i + 1] + _CH - 1) // _CH

        def do_chunk(k, i=i):
            rv = rl_ref[pl.ds(k, 1), :] - i * _T
            cv = cl_ref[pl.ds(k, 1), :]
            d_oh = (row_iota == rv).astype(jnp.float8_e4m3fn)
            s_oh = (col_iota == cv).astype(jnp.float8_e4m3fn)
            for kt in range(_CH // 256):
                d_t = d_oh[:, kt * 256:(kt + 1) * 256]
                s_t = s_oh[:, kt * 256:(kt + 1) * 256]
                for m in (0, 1):
                    pltpu.matmul_push_rhs(d_t, staging_register=0,
                                          mxu_index=m, transpose=True)
                    pltpu.matmul_acc_lhs(0, s_t[m * half:(m + 1) * half, :],
                                         mxu_index=m, load_staged_rhs=0)

        def body(p_idx, _, do_chunk=do_chunk):
            do_chunk(2 * p_idx)
            do_chunk(2 * p_idx + 1)
            return _

        jax.lax.fori_loop(k0 // 2, (k1 + 1) // 2, body, 0)
        acc = jnp.concatenate(
            [pltpu.matmul_pop(0, (half, _T), jnp.float32, m)
             for m in (0, 1)], axis=0)
        at_ref[:, t * _T:(t + 1) * _T] = acc.astype(at_ref.dtype)
        deg_ref[t * _T:(t + 1) * _T, :] = jnp.sum(
            acc, axis=0, keepdims=True).reshape(_T, 1)


def _xw_kernel(x_ref, w_ref, dis_ref, o_ref):
    i = pl.program_id(0)
    x = x_ref[...].astype(jnp.bfloat16)
    w = w_ref[...].astype(jnp.bfloat16)
    dis_i = dis_ref[pl.ds(i * _T, _T), :]
    o_ref[...] = (dis_i * jnp.dot(
        x, w, preferred_element_type=jnp.float32
    )).astype(jnp.bfloat16)


def _h_kernel(at_ref, p_ref, w2_ref, b1_ref, dis_ref, q_ref):
    i = pl.program_id(0)
    h = jax.lax.dot_general(
        at_ref[...].astype(jnp.bfloat16), p_ref[...],
        (((0,), (0,)), ((), ())),
        preferred_element_type=jnp.float32)
    p_i = p_ref[pl.ds(i * _TB, _TB), :].astype(jnp.float32)
    dis_i = dis_ref[pl.ds(i * _TB, _TB), :]
    h = jnp.maximum(dis_i * (h + p_i) + b1_ref[...], 0.0)
    q = jnp.dot(h.astype(jnp.bfloat16), w2_ref[...],
                preferred_element_type=jnp.float32)
    q_ref[...] = (dis_i * q).astype(jnp.bfloat16)


def _out_kernel(at_ref, q_ref, b2_ref, dis_ref, o_ref):
    i = pl.program_id(0)
    o = jax.lax.dot_general(
        at_ref[...].astype(jnp.bfloat16), q_ref[...],
        (((0,), (0,)), ((), ())),
        preferred_element_type=jnp.float32)
    q_i = q_ref[pl.ds(i * _TB, _TB), :].astype(jnp.float32)
    dis_i = dis_ref[pl.ds(i * _TB, _TB), :]
    c = o_ref.shape[1]
    o_ref[...] = (dis_i * (o + q_i) + b2_ref[...])[:, :c]


def kernel(x, edge_index, w1, b1, w2, b2):
    n, f_in = x.shape
    hidden = w1.shape[1]
    c = w2.shape[1]
    c_pad = max(128, ((c + 127) // 128) * 128)
    e = edge_index.shape[1]
    n_tiles = n // _T
    e_pad = ((e + 4 * _CH - 1) // (4 * _CH)) * (4 * _CH)

    src = edge_index[0]
    dst = edge_index[1]
    lin = jnp.sort(jnp.concatenate(
        [dst * n + src, jnp.full((e_pad - e,), n * n, jnp.int32)]))
    tile_of_edge = dst // _T
    bounds = jnp.concatenate([
        jnp.zeros((1,), jnp.int32),
        jnp.cumsum(jnp.sum(
            tile_of_edge[None, :] == jnp.arange(n_tiles)[:, None],
            axis=1, dtype=jnp.int32))])
    rl = (lin // n).reshape(e_pad // _CH, _CH)
    cl = (lin % n).reshape(e_pad // _CH, _CH)

    w2p = jnp.zeros((hidden, c_pad), jnp.bfloat16).at[:, :c].set(
        w2.astype(jnp.bfloat16))
    b2p = jnp.zeros((1, c_pad), jnp.float32).at[:, :c].set(b2)

    grid = (n_tiles,)
    par = pltpu.CompilerParams(dimension_semantics=("parallel",))

    a_t, deg = pl.pallas_call(
        _build_a_kernel,
        out_shape=(jax.ShapeDtypeStruct((n, n), jnp.float8_e4m3fn),
                   jax.ShapeDtypeStruct((n, 1), jnp.float32)),
        grid=(n_tiles // 2,),
        in_specs=[
            pl.BlockSpec(memory_space=pltpu.SMEM),
            pl.BlockSpec((e_pad // _CH, _CH), lambda i: (0, 0)),
            pl.BlockSpec((e_pad // _CH, _CH), lambda i: (0, 0)),
        ],
        out_specs=(pl.BlockSpec((n, 2 * _T), lambda i: (0, i)),
                   pl.BlockSpec((2 * _T, 1), lambda i: (i, 0))),
        compiler_params=par,
    )(bounds, rl, cl)

    dis = 1.0 / jnp.sqrt(deg + 1.0)

    p_s = pl.pallas_call(
        _xw_kernel,
        out_shape=jax.ShapeDtypeStruct((n, hidden), jnp.bfloat16),
        grid=grid,
        in_specs=[
            pl.BlockSpec((_T, f_in), lambda i: (i, 0)),
            pl.BlockSpec((f_in, hidden), lambda i: (0, 0)),
            pl.BlockSpec((n, 1), lambda i: (0, 0)),
        ],
        out_specs=pl.BlockSpec((_T, hidden), lambda i: (i, 0)),
        compiler_params=par,
    )(x, w1, dis)

    grid_b = (n // _TB,)
    q = pl.pallas_call(
        _h_kernel,
        out_shape=jax.ShapeDtypeStruct((n, c_pad), jnp.bfloat16),
        grid=grid_b,
        in_specs=[
            pl.BlockSpec((n, _TB), lambda i: (0, i)),
            pl.BlockSpec((n, hidden), lambda i: (0, 0)),
            pl.BlockSpec((hidden, c_pad), lambda i: (0, 0)),
            pl.BlockSpec((1, hidden), lambda i: (0, 0)),
            pl.BlockSpec((n, 1), lambda i: (0, 0)),
        ],
        out_specs=pl.BlockSpec((_TB, c_pad), lambda i: (i, 0)),
        compiler_params=par,
    )(a_t, p_s, w2p, b1, dis)

    out = pl.pallas_call(
        _out_kernel,
        out_shape=jax.ShapeDtypeStruct((n, c), jnp.float32),
        grid=grid_b,
        in_specs=[
            pl.BlockSpec((n, _TB), lambda i: (0, i)),
            pl.BlockSpec((n, c_pad), lambda i: (0, 0)),
            pl.BlockSpec((1, c_pad), lambda i: (0, 0)),
            pl.BlockSpec((n, 1), lambda i: (0, 0)),
        ],
        out_specs=pl.BlockSpec((_TB, c), lambda i: (i, 0)),
        compiler_params=par,
    )(a_t, q, b2p, dis)

    return out


# speedup vs baseline: 1.0151x; 1.0151x over previous
import jax
import jax.numpy as jnp
from jax.experimental import pallas as pl
from jax.experimental.pallas import tpu as pltpu

_T = 256
_TB = 512
_CH = 512


def _build_a_kernel(bounds_ref, rl_ref, cl_ref, at_ref, deg_ref):
    g = pl.program_id(0)
    n = at_ref.shape[0]
    half = n // 2

    row_iota = jax.lax.broadcasted_iota(jnp.int32, (_T, _CH), 0)
    col_iota = jax.lax.broadcasted_iota(jnp.int32, (n, _CH), 0)

    junk = jnp.concatenate(
        [pltpu.matmul_pop(0, (half, _T), jnp.float32, m) for m in (0, 1)],
        axis=0)
    at_ref[:, :_T] = junk.astype(at_ref.dtype)

    for t in range(2):
        i = 2 * g + t
        k0 = bounds_ref[i] // _CH
        k1 = (bounds_ref[i + 1] + _CH - 1) // _CH

        def do_chunk(k, i=i):
            rv = rl_ref[pl.ds(k, 1), :] - i * _T
            cv = cl_ref[pl.ds(k, 1), :]
            d_oh = (row_iota == rv).astype(jnp.float8_e4m3fn)
            s_oh = (col_iota == cv).astype(jnp.float8_e4m3fn)
            for kt in range(_CH // 256):
                d_t = d_oh[:, kt * 256:(kt + 1) * 256]
                s_t = s_oh[:, kt * 256:(kt + 1) * 256]
                for m in (0, 1):
                    pltpu.matmul_push_rhs(d_t, staging_register=0,
                                          mxu_index=m, transpose=True)
                    pltpu.matmul_acc_lhs(0, s_t[m * half:(m + 1) * half, :],
                                         mxu_index=m, load_staged_rhs=0)

        def body(p_idx, _, do_chunk=do_chunk):
            do_chunk(2 * p_idx)
            do_chunk(2 * p_idx + 1)
            return _

        jax.lax.fori_loop(k0 // 2, (k1 + 1) // 2, body, 0)
        acc = jnp.concatenate(
            [pltpu.matmul_pop(0, (half, _T), jnp.float32, m)
             for m in (0, 1)], axis=0)
        at_ref[:, t * _T:(t + 1) * _T] = acc.astype(at_ref.dtype)
        deg_ref[:, t * _T:(t + 1) * _T] = jnp.sum(acc, axis=0,
                                                  keepdims=True)


def _xw_kernel(x_ref, w_ref, dis_ref, o_ref):
    i = pl.program_id(0)
    x = x_ref[...].astype(jnp.bfloat16)
    w = w_ref[...].astype(jnp.bfloat16)
    dis_i = dis_ref[pl.ds(i * _T, _T), :]
    o_ref[...] = (dis_i * jnp.dot(
        x, w, preferred_element_type=jnp.float32
    )).astype(jnp.bfloat16)


def _h_kernel(at_ref, p_ref, w2_ref, b1_ref, dis_ref, q_ref):
    i = pl.program_id(0)
    h = jax.lax.dot_general(
        at_ref[...].astype(jnp.bfloat16), p_ref[...],
        (((0,), (0,)), ((), ())),
        preferred_element_type=jnp.float32)
    p_i = p_ref[pl.ds(i * _TB, _TB), :].astype(jnp.float32)
    dis_i = dis_ref[pl.ds(i * _TB, _TB), :]
    h = jnp.maximum(dis_i * (h + p_i) + b1_ref[...], 0.0)
    q = jnp.dot(h.astype(jnp.bfloat16), w2_ref[...],
                preferred_element_type=jnp.float32)
    q_ref[...] = (dis_i * q).astype(jnp.bfloat16)


def _out_kernel(at_ref, q_ref, b2_ref, dis_ref, o_ref):
    i = pl.program_id(0)
    o = jax.lax.dot_general(
        at_ref[...].astype(jnp.bfloat16), q_ref[...],
        (((0,), (0,)), ((), ())),
        preferred_element_type=jnp.float32)
    q_i = q_ref[pl.ds(i * _TB, _TB), :].astype(jnp.float32)
    dis_i = dis_ref[pl.ds(i * _TB, _TB), :]
    c = o_ref.shape[1]
    o_ref[...] = (dis_i * (o + q_i) + b2_ref[...])[:, :c]


def kernel(x, edge_index, w1, b1, w2, b2):
    n, f_in = x.shape
    hidden = w1.shape[1]
    c = w2.shape[1]
    c_pad = max(128, ((c + 127) // 128) * 128)
    e = edge_index.shape[1]
    n_tiles = n // _T
    e_pad = ((e + 4 * _CH - 1) // (4 * _CH)) * (4 * _CH)

    src = edge_index[0]
    dst = edge_index[1]
    lin = jnp.sort(jnp.concatenate(
        [dst * n + src, jnp.full((e_pad - e,), n * n, jnp.int32)]))
    tile_of_edge = dst // _T
    bounds = jnp.concatenate([
        jnp.zeros((1,), jnp.int32),
        jnp.cumsum(jnp.sum(
            tile_of_edge[None, :] == jnp.arange(n_tiles)[:, None],
            axis=1, dtype=jnp.int32))])
    rl = (lin // n).reshape(e_pad // _CH, _CH)
    cl = (lin % n).reshape(e_pad // _CH, _CH)

    w2p = jnp.zeros((hidden, c_pad), jnp.bfloat16).at[:, :c].set(
        w2.astype(jnp.bfloat16))
    b2p = jnp.zeros((1, c_pad), jnp.float32).at[:, :c].set(b2)

    grid = (n_tiles,)
    par = pltpu.CompilerParams(dimension_semantics=("parallel",))

    a_t, deg = pl.pallas_call(
        _build_a_kernel,
        out_shape=(jax.ShapeDtypeStruct((n, n), jnp.float8_e4m3fn),
                   jax.ShapeDtypeStruct((1, n), jnp.float32)),
        grid=(n_tiles // 2,),
        in_specs=[
            pl.BlockSpec(memory_space=pltpu.SMEM),
            pl.BlockSpec((e_pad // _CH, _CH), lambda i: (0, 0)),
            pl.BlockSpec((e_pad // _CH, _CH), lambda i: (0, 0)),
        ],
        out_specs=(pl.BlockSpec((n, 2 * _T), lambda i: (0, i)),
                   pl.BlockSpec((1, 2 * _T), lambda i: (0, i))),
        compiler_params=par,
    )(bounds, rl, cl)

    dis = (1.0 / jnp.sqrt(deg + 1.0)).reshape(n, 1)

    p_s = pl.pallas_call(
        _xw_kernel,
        out_shape=jax.ShapeDtypeStruct((n, hidden), jnp.bfloat16),
        grid=grid,
        in_specs=[
            pl.BlockSpec((_T, f_in), lambda i: (i, 0)),
            pl.BlockSpec((f_in, hidden), lambda i: (0, 0)),
            pl.BlockSpec((n, 1), lambda i: (0, 0)),
        ],
        out_specs=pl.BlockSpec((_T, hidden), lambda i: (i, 0)),
        compiler_params=par,
    )(x, w1, dis)

    grid_b = (n // _TB,)
    q = pl.pallas_call(
        _h_kernel,
        out_shape=jax.ShapeDtypeStruct((n, c_pad), jnp.bfloat16),
        grid=grid_b,
        in_specs=[
            pl.BlockSpec((n, _TB), lambda i: (0, i)),
            pl.BlockSpec((n, hidden), lambda i: (0, 0)),
            pl.BlockSpec((hidden, c_pad), lambda i: (0, 0)),
            pl.BlockSpec((1, hidden), lambda i: (0, 0)),
            pl.BlockSpec((n, 1), lambda i: (0, 0)),
        ],
        out_specs=pl.BlockSpec((_TB, c_pad), lambda i: (i, 0)),
        compiler_params=par,
    )(a_t, p_s, w2p, b1, dis)

    out = pl.pallas_call(
        _out_kernel,
        out_shape=jax.ShapeDtypeStruct((n, c), jnp.float32),
        grid=grid_b,
        in_specs=[
            pl.BlockSpec((n, _TB), lambda i: (0, i)),
            pl.BlockSpec((n, c_pad), lambda i: (0, 0)),
            pl.BlockSpec((1, c_pad), lambda i: (0, 0)),
            pl.BlockSpec((n, 1), lambda i: (0, 0)),
        ],
        out_specs=pl.BlockSpec((_TB, c), lambda i: (i, 0)),
        compiler_params=par,
    )(a_t, q, b2p, dis)

    return out
